# baseline (device time: 26122 ns/iter reference)
import jax
import jax.numpy as jnp
from jax import lax
from jax.experimental import pallas as pl
from jax.experimental.pallas import tpu as pltpu

N_DEV = 4
NROW = 2
NCOL = 2


def kernel(x):
    m, n = x.shape
    hn = n // 2
    cw = hn // NCOL
    gm = m // NROW
    hm2 = gm // 2
    qm2 = gm // 4

    def body(
        x_ref,
        out_ref,
        xv_ref,
        xbs_ref,
        st1a,
        st1b,
        st2a,
        st2b,
        red1a,
        red1b,
        copy_sems,
        send_sems,
        recv_sems,
    ):
        my = lax.axis_index("i")
        b0 = my & 1
        b1 = my >> 1
        pa1 = my ^ 1
        pa2 = my ^ 3

        in_copies = []
        for rc in range(NROW):
            cp = pltpu.make_async_copy(
                x_ref.at[pl.ds(rc * gm, gm), :],
                xv_ref.at[pl.ds(rc * gm, gm), :],
                copy_sems.at[rc],
            )
            cp.start()
            in_copies.append(cp)

        barrier_sem = pltpu.get_barrier_semaphore()
        for nbr in [pa1, pa2]:
            pl.semaphore_signal(
                barrier_sem, inc=1,
                device_id=(nbr,), device_id_type=pl.DeviceIdType.MESH,
            )
        pl.semaphore_wait(barrier_sem, 2)

        keep_lo_a = b0 == b1
        off_keep_a = jnp.where(keep_lo_a, 0, hm2)
        off_send_a = jnp.where(keep_lo_a, hm2, 0)
        keep_lo_b = b1 == 0
        off_keep_b = jnp.where(keep_lo_b, 0, hm2)
        off_send_b = jnp.where(keep_lo_b, hm2, 0)
        keep_first_a = b1 == 0
        k2a = jnp.where(keep_first_a, 0, qm2)
        s2a = jnp.where(keep_first_a, qm2, 0)
        keep_first_b = b0 == 0
        k2b = jnp.where(keep_first_b, 0, qm2)
        s2b = jnp.where(keep_first_b, qm2, 0)
        qoff_a = off_keep_a + k2a
        qoff_b = off_keep_b + k2b
        qoff2_a = off_keep_a + s2a
        qoff2_b = off_keep_b + s2b

        sends = []

        def exch(idx, src, dst, tgt):
            r = pltpu.make_async_remote_copy(
                src_ref=src,
                dst_ref=dst,
                send_sem=send_sems.at[idx],
                recv_sem=recv_sems.at[idx],
                device_id=(tgt,),
                device_id_type=pl.DeviceIdType.MESH,
            )
            r.start()
            sends.append(r)
            return r

        units = [(rc, cc) for rc in range(NROW) for cc in range(NCOL)]

        def ac(cc):
            return pl.ds(cc * cw, cw)

        def bc(cc):
            return pl.ds(hn + cc * cw, cw)

        def sem(u, e):
            return u * 8 + e

        r1 = []
        for u, (rc, cc) in enumerate(units):
            rb = rc * gm
            srow = pl.ds(rc * hm2, hm2)
            if cc == 0:
                in_copies[rc].wait()
            xbs_ref[srow, ac(cc)] = xv_ref[
                pl.ds(rb + off_send_a, hm2), ac(cc)
            ].astype(jnp.bfloat16)
            xbs_ref[srow, bc(cc)] = xv_ref[
                pl.ds(rb + off_send_b, hm2), bc(cc)
            ].astype(jnp.bfloat16)
            r1.append((
                exch(sem(u, 0), xbs_ref.at[srow, ac(cc)],
                     st1a.at[srow, ac(cc)], pa1),
                exch(sem(u, 1), xbs_ref.at[srow, bc(cc)],
                     st1b.at[srow, ac(cc)], pa2),
            ))

        r2 = [None] * len(units)
        for u, (rc, cc) in enumerate(units):
            rb = rc * gm
            srow = pl.ds(rc * hm2, hm2)
            ra, rbx = r1[u]
            ra.wait_recv()
            rbx.wait_recv()
            red1a[srow, ac(cc)] = (
                xv_ref[pl.ds(rb + off_keep_a, hm2), ac(cc)]
                + st1a[srow, ac(cc)].astype(jnp.float32)
            ).astype(jnp.bfloat16)
            red1b[srow, ac(cc)] = (
                xv_ref[pl.ds(rb + off_keep_b, hm2), bc(cc)]
                + st1b[srow, ac(cc)].astype(jnp.float32)
            ).astype(jnp.bfloat16)
            qrow = pl.ds(rc * qm2, qm2)
            r2[u] = (
                exch(sem(u, 2), red1a.at[pl.ds(rc * hm2 + s2a, qm2), ac(cc)],
                     st2a.at[qrow, ac(cc)], pa2),
                exch(sem(u, 3), red1b.at[pl.ds(rc * hm2 + s2b, qm2), ac(cc)],
                     st2b.at[qrow, ac(cc)], pa1),
            )

        r3 = [None] * len(units)
        r4i = [None] * len(units)
        for u, (rc, cc) in enumerate(units):
            rb = rc * gm
            qrow = pl.ds(rc * qm2, qm2)
            ra, rbx = r2[u]
            ra.wait_recv()
            rbx.wait_recv()
            out_ref[pl.ds(rb + qoff_a, qm2), ac(cc)] = (
                red1a[pl.ds(rc * hm2 + k2a, qm2), ac(cc)].astype(jnp.float32)
                + st2a[qrow, ac(cc)].astype(jnp.float32)
            ).astype(jnp.bfloat16)
            out_ref[pl.ds(rb + qoff_b, qm2), bc(cc)] = (
                red1b[pl.ds(rc * hm2 + k2b, qm2), ac(cc)].astype(jnp.float32)
                + st2b[qrow, ac(cc)].astype(jnp.float32)
            ).astype(jnp.bfloat16)
            r3[u] = (
                exch(sem(u, 4), out_ref.at[pl.ds(rb + qoff_a, qm2), ac(cc)],
                     out_ref.at[pl.ds(rb + qoff_a, qm2), ac(cc)], pa2),
                exch(sem(u, 5), out_ref.at[pl.ds(rb + qoff_b, qm2), bc(cc)],
                     out_ref.at[pl.ds(rb + qoff_b, qm2), bc(cc)], pa1),
            )
            r4i[u] = (
                exch(sem(u, 6), out_ref.at[pl.ds(rb + qoff_a, qm2), ac(cc)],
                     out_ref.at[pl.ds(rb + qoff_a, qm2), ac(cc)], pa1),
                exch(sem(u, 7), out_ref.at[pl.ds(rb + qoff_b, qm2), bc(cc)],
                     out_ref.at[pl.ds(rb + qoff_b, qm2), bc(cc)], pa2),
            )

        r4ii = [None] * len(units)
        for u, (rc, cc) in enumerate(units):
            rb = rc * gm
            ra, rbx = r3[u]
            ra.wait_recv()
            rbx.wait_recv()
            s1a_r, s1b_r = r1[u]
            s1a_r.wait_send()
            s1b_r.wait_send()
            r4ii[u] = (
                exch(sem(u, 0), out_ref.at[pl.ds(rb + qoff2_a, qm2), ac(cc)],
                     out_ref.at[pl.ds(rb + qoff2_a, qm2), ac(cc)], pa1),
                exch(sem(u, 1), out_ref.at[pl.ds(rb + qoff2_b, qm2), bc(cc)],
                     out_ref.at[pl.ds(rb + qoff2_b, qm2), bc(cc)], pa2),
            )

        for u in range(len(units)):
            ra, rbx = r4i[u]
            ra.wait_recv()
            rbx.wait_recv()
            ra2, rbx2 = r4ii[u]
            ra2.wait_recv()
            rbx2.wait_recv()

        r1_ids = {id(r) for pair in r1 for r in pair}
        for r in sends:
            if id(r) not in r1_ids:
                r.wait_send()

    nu = NROW * NCOL
    return pl.pallas_call(
        body,
        out_shape=jax.ShapeDtypeStruct((m, n), jnp.bfloat16),
        in_specs=[pl.BlockSpec(memory_space=pl.ANY)],
        out_specs=pl.BlockSpec(memory_space=pltpu.VMEM),
        scratch_shapes=[
            pltpu.VMEM((m, n), jnp.float32),
            pltpu.VMEM((NROW * hm2, n), jnp.bfloat16),
            pltpu.VMEM((NROW * hm2, hn), jnp.bfloat16),
            pltpu.VMEM((NROW * hm2, hn), jnp.bfloat16),
            pltpu.VMEM((NROW * qm2, hn), jnp.bfloat16),
            pltpu.VMEM((NROW * qm2, hn), jnp.bfloat16),
            pltpu.VMEM((NROW * hm2, hn), jnp.bfloat16),
            pltpu.VMEM((NROW * hm2, hn), jnp.bfloat16),
            pltpu.SemaphoreType.DMA((NROW,)),
            pltpu.SemaphoreType.DMA((nu * 8,)),
            pltpu.SemaphoreType.DMA((nu * 8,)),
        ],
        compiler_params=pltpu.CompilerParams(collective_id=0),
    )(x)


# device time: 24755 ns/iter; 1.0552x vs baseline; 1.0552x over previous
import jax
import jax.numpy as jnp
from jax import lax
from jax.experimental import pallas as pl
from jax.experimental.pallas import tpu as pltpu

N_DEV = 4
NROW = 2
NCOL = 2


def kernel(x):
    m, n = x.shape
    hn = n // 2
    cw = hn // NCOL
    gm = m // NROW
    hm2 = gm // 2
    qm2 = gm // 4

    def body(
        x_ref,
        out_ref,
        xbs_ref,
        st1a,
        st1b,
        st2a,
        st2b,
        red1a,
        red1b,
        send_sems,
        recv_sems,
    ):
        my = lax.axis_index("i")
        b0 = my & 1
        b1 = my >> 1
        pa1 = my ^ 1
        pa2 = my ^ 3

        barrier_sem = pltpu.get_barrier_semaphore()
        for nbr in [pa1, pa2]:
            pl.semaphore_signal(
                barrier_sem, inc=1,
                device_id=(nbr,), device_id_type=pl.DeviceIdType.MESH,
            )
        pl.semaphore_wait(barrier_sem, 2)

        keep_lo_a = b0 == b1
        off_keep_a = jnp.where(keep_lo_a, 0, hm2)
        off_send_a = jnp.where(keep_lo_a, hm2, 0)
        keep_lo_b = b1 == 0
        off_keep_b = jnp.where(keep_lo_b, 0, hm2)
        off_send_b = jnp.where(keep_lo_b, hm2, 0)
        keep_first_a = b1 == 0
        k2a = jnp.where(keep_first_a, 0, qm2)
        s2a = jnp.where(keep_first_a, qm2, 0)
        keep_first_b = b0 == 0
        k2b = jnp.where(keep_first_b, 0, qm2)
        s2b = jnp.where(keep_first_b, qm2, 0)
        qoff_a = off_keep_a + k2a
        qoff_b = off_keep_b + k2b
        qoff2_a = off_keep_a + s2a
        qoff2_b = off_keep_b + s2b

        sends = []

        def exch(idx, src, dst, tgt):
            r = pltpu.make_async_remote_copy(
                src_ref=src,
                dst_ref=dst,
                send_sem=send_sems.at[idx],
                recv_sem=recv_sems.at[idx],
                device_id=(tgt,),
                device_id_type=pl.DeviceIdType.MESH,
            )
            r.start()
            sends.append(r)
            return r

        units = [(rc, cc) for rc in range(NROW) for cc in range(NCOL)]

        def ac(cc):
            return pl.ds(cc * cw, cw)

        def bc(cc):
            return pl.ds(hn + cc * cw, cw)

        def sem(u, e):
            return u * 8 + e

        r1 = []
        for u, (rc, cc) in enumerate(units):
            rb = rc * gm
            srow = pl.ds(rc * hm2, hm2)
            xbs_ref[srow, ac(cc)] = x_ref[
                pl.ds(rb + off_send_a, hm2), ac(cc)
            ].astype(jnp.bfloat16)
            xbs_ref[srow, bc(cc)] = x_ref[
                pl.ds(rb + off_send_b, hm2), bc(cc)
            ].astype(jnp.bfloat16)
            r1.append((
                exch(sem(u, 0), xbs_ref.at[srow, ac(cc)],
                     st1a.at[srow, ac(cc)], pa1),
                exch(sem(u, 1), xbs_ref.at[srow, bc(cc)],
                     st1b.at[srow, ac(cc)], pa2),
            ))

        r2 = [None] * len(units)
        for u, (rc, cc) in enumerate(units):
            rb = rc * gm
            srow = pl.ds(rc * hm2, hm2)
            ra, rbx = r1[u]
            ra.wait_recv()
            rbx.wait_recv()
            red1a[srow, ac(cc)] = (
                x_ref[pl.ds(rb + off_keep_a, hm2), ac(cc)]
                + st1a[srow, ac(cc)].astype(jnp.float32)
            ).astype(jnp.bfloat16)
            red1b[srow, ac(cc)] = (
                x_ref[pl.ds(rb + off_keep_b, hm2), bc(cc)]
                + st1b[srow, ac(cc)].astype(jnp.float32)
            ).astype(jnp.bfloat16)
            qrow = pl.ds(rc * qm2, qm2)
            r2[u] = (
                exch(sem(u, 2), red1a.at[pl.ds(rc * hm2 + s2a, qm2), ac(cc)],
                     st2a.at[qrow, ac(cc)], pa2),
                exch(sem(u, 3), red1b.at[pl.ds(rc * hm2 + s2b, qm2), ac(cc)],
                     st2b.at[qrow, ac(cc)], pa1),
            )

        r3 = [None] * len(units)
        r4i = [None] * len(units)
        for u, (rc, cc) in enumerate(units):
            rb = rc * gm
            qrow = pl.ds(rc * qm2, qm2)
            ra, rbx = r2[u]
            ra.wait_recv()
            rbx.wait_recv()
            out_ref[pl.ds(rb + qoff_a, qm2), ac(cc)] = (
                red1a[pl.ds(rc * hm2 + k2a, qm2), ac(cc)].astype(jnp.float32)
                + st2a[qrow, ac(cc)].astype(jnp.float32)
            ).astype(jnp.bfloat16)
            out_ref[pl.ds(rb + qoff_b, qm2), bc(cc)] = (
                red1b[pl.ds(rc * hm2 + k2b, qm2), ac(cc)].astype(jnp.float32)
                + st2b[qrow, ac(cc)].astype(jnp.float32)
            ).astype(jnp.bfloat16)
            r3[u] = (
                exch(sem(u, 4), out_ref.at[pl.ds(rb + qoff_a, qm2), ac(cc)],
                     out_ref.at[pl.ds(rb + qoff_a, qm2), ac(cc)], pa2),
                exch(sem(u, 5), out_ref.at[pl.ds(rb + qoff_b, qm2), bc(cc)],
                     out_ref.at[pl.ds(rb + qoff_b, qm2), bc(cc)], pa1),
            )
            r4i[u] = (
                exch(sem(u, 6), out_ref.at[pl.ds(rb + qoff_a, qm2), ac(cc)],
                     out_ref.at[pl.ds(rb + qoff_a, qm2), ac(cc)], pa1),
                exch(sem(u, 7), out_ref.at[pl.ds(rb + qoff_b, qm2), bc(cc)],
                     out_ref.at[pl.ds(rb + qoff_b, qm2), bc(cc)], pa2),
            )

        r4ii = [None] * len(units)
        for u, (rc, cc) in enumerate(units):
            rb = rc * gm
            ra, rbx = r3[u]
            ra.wait_recv()
            rbx.wait_recv()
            s1a_r, s1b_r = r1[u]
            s1a_r.wait_send()
            s1b_r.wait_send()
            r4ii[u] = (
                exch(sem(u, 0), out_ref.at[pl.ds(rb + qoff2_a, qm2), ac(cc)],
                     out_ref.at[pl.ds(rb + qoff2_a, qm2), ac(cc)], pa1),
                exch(sem(u, 1), out_ref.at[pl.ds(rb + qoff2_b, qm2), bc(cc)],
                     out_ref.at[pl.ds(rb + qoff2_b, qm2), bc(cc)], pa2),
            )

        for u in range(len(units)):
            ra, rbx = r4i[u]
            ra.wait_recv()
            rbx.wait_recv()
            ra2, rbx2 = r4ii[u]
            ra2.wait_recv()
            rbx2.wait_recv()

        r1_ids = {id(r) for pair in r1 for r in pair}
        for r in sends:
            if id(r) not in r1_ids:
                r.wait_send()

    nu = NROW * NCOL
    return pl.pallas_call(
        body,
        out_shape=jax.ShapeDtypeStruct((m, n), jnp.bfloat16),
        in_specs=[pl.BlockSpec(memory_space=pltpu.VMEM)],
        out_specs=pl.BlockSpec(memory_space=pltpu.VMEM),
        scratch_shapes=[
            pltpu.VMEM((NROW * hm2, n), jnp.bfloat16),
            pltpu.VMEM((NROW * hm2, hn), jnp.bfloat16),
            pltpu.VMEM((NROW * hm2, hn), jnp.bfloat16),
            pltpu.VMEM((NROW * qm2, hn), jnp.bfloat16),
            pltpu.VMEM((NROW * qm2, hn), jnp.bfloat16),
            pltpu.VMEM((NROW * hm2, hn), jnp.bfloat16),
            pltpu.VMEM((NROW * hm2, hn), jnp.bfloat16),
            pltpu.SemaphoreType.DMA((nu * 8,)),
            pltpu.SemaphoreType.DMA((nu * 8,)),
        ],
        compiler_params=pltpu.CompilerParams(collective_id=0),
    )(x)
